# baseline (speedup 1.0000x reference)
import functools

import jax
import jax.numpy as jnp
from jax import lax
from jax.experimental import pallas as pl
from jax.experimental.pallas import tpu as pltpu
from jax.experimental.pallas import tpu_sc as plsc

N = 10000
E = 160000
H = 128
NC = 2
NS = 16
NW = NC * NS
EPW = E // NW
CK = 128
NCH = 40
PAD = NCH * CK - EPW
NP = 10240
RPS = NP // NS
ZR = 128


def _mesh():
    return plsc.VectorSubcoreMesh(core_axis_name="c", subcore_axis_name="s",
                                  num_cores=NC, num_subcores=NS)


def _segsum_body(hp_hbm, src_hbm, dst_hbm, out_hbm,
                 src_v, dst_v, buf0, buf1, gsem0, gsem1, acc_sh):
    c = lax.axis_index("c")
    s = lax.axis_index("s")
    w = c * NS + s

    pltpu.sync_copy(src_hbm.at[pl.ds(w * NCH, NCH)], src_v)
    pltpu.sync_copy(dst_hbm.at[pl.ds(w * NCH, NCH)], dst_v)

    def fill(r, carry):
        for t in range(H // 16):
            buf0[r, pl.ds(t * 16, 16)] = jnp.zeros((16,), jnp.float32)
        return carry
    lax.fori_loop(0, ZR, fill, 0)

    def _wait(ref, sem):
        pltpu.make_async_copy(hp_hbm.at[pl.ds(0, CK)], ref, sem).wait()

    for t in range(RPS // ZR):
        pltpu.async_copy(buf0, acc_sh.at[pl.ds(s * RPS + t * ZR, ZR)], gsem0)
    for t in range(RPS // ZR):
        _wait(buf0, gsem0)
    pltpu.async_copy(hp_hbm.at[src_v.at[0]], buf0, gsem0)
    pltpu.async_copy(hp_hbm.at[src_v.at[1]], buf1, gsem1)
    plsc.subcore_barrier()

    def pair(j2, carry):
        j = 2 * j2
        _wait(buf0, gsem0)
        pltpu.sync_copy(buf0, acc_sh.at[dst_v.at[j]], add=True)

        @pl.when(j + 2 < NCH)
        def _():
            pltpu.async_copy(hp_hbm.at[src_v.at[j + 2]], buf0, gsem0)
        _wait(buf1, gsem1)
        pltpu.sync_copy(buf1, acc_sh.at[dst_v.at[j + 1]], add=True)

        @pl.when(j + 3 < NCH)
        def _():
            pltpu.async_copy(hp_hbm.at[src_v.at[j + 3]], buf1, gsem1)
        return carry
    lax.fori_loop(0, NCH // 2, pair, 0)
    plsc.subcore_barrier()

    pltpu.sync_copy(acc_sh.at[pl.ds(s * RPS, RPS)],
                    out_hbm.at[c, pl.ds(s * RPS, RPS)])


@functools.lru_cache(maxsize=None)
def _sc_kernels():
    segsum = pl.kernel(
        _segsum_body,
        out_type=jax.ShapeDtypeStruct((NC, NP, H), jnp.float32),
        mesh=_mesh(),
        scratch_types=[
            pltpu.VMEM((NCH, CK), jnp.int32),
            pltpu.VMEM((NCH, CK), jnp.int32),
            pltpu.VMEM((CK, H), jnp.float32),
            pltpu.VMEM((CK, H), jnp.float32),
            pltpu.SemaphoreType.DMA,
            pltpu.SemaphoreType.DMA,
            pltpu.VMEM_SHARED((NP, H), jnp.float32),
        ],
    )
    return segsum


def _mm_body(x_ref, w_ref, o_ref):
    o_ref[...] = jnp.dot(x_ref[...], w_ref[...],
                         preferred_element_type=jnp.float32)


def _matmul(x, w, bm):
    m, k = x.shape
    n = w.shape[1]
    return pl.pallas_call(
        _mm_body,
        grid=(m // bm,),
        in_specs=[pl.BlockSpec((bm, k), lambda i: (i, 0)),
                  pl.BlockSpec((k, n), lambda i: (0, 0))],
        out_specs=pl.BlockSpec((bm, n), lambda i: (i, 0)),
        out_shape=jax.ShapeDtypeStruct((m, n), jnp.float32),
    )(x, w)


def _t1_body(dp_ref, h1_ref, dis_ref, h1p_ref):
    deg = 1.0 + dp_ref[0, 0:N, 0:1] + dp_ref[1, 0:N, 0:1]
    dis = lax.rsqrt(deg)
    dis_ref[...] = dis
    h1p_ref[...] = h1_ref[...] * dis


def _t1(deg_parts, h1):
    return pl.pallas_call(
        _t1_body,
        out_shape=[jax.ShapeDtypeStruct((N, 1), jnp.float32),
                   jax.ShapeDtypeStruct((N, H), jnp.float32)],
    )(deg_parts, h1)


def _t2_body(sp_ref, hp_ref, dis_ref, b_ref, g_ref, be_ref, w_ref, o_ref):
    dis = dis_ref[...]
    pre = (sp_ref[0, 0:N, :] + sp_ref[1, 0:N, :] + hp_ref[...]) * dis \
        + b_ref[...]
    m = jnp.mean(pre, axis=0, keepdims=True)
    v = jnp.mean((pre - m) ** 2, axis=0, keepdims=True)
    act = jnp.maximum((pre - m) * lax.rsqrt(v + 1e-5) * g_ref[...]
                      + be_ref[...], 0.0)
    o_ref[...] = jnp.dot(act, w_ref[...],
                         preferred_element_type=jnp.float32) * dis


def _t2(sp, hp, dis, b, g, be, w_next):
    return pl.pallas_call(
        _t2_body,
        out_shape=jax.ShapeDtypeStruct((N, H), jnp.float32),
    )(sp, hp, dis, b.reshape(1, H), g.reshape(1, H), be.reshape(1, H),
      w_next)


def _t3_body(sp_ref, hp_ref, dis_ref, b_ref, g_ref, be_ref, wfc_ref,
             bfc_ref, o_ref):
    pre = (sp_ref[0, 0:N, :] + sp_ref[1, 0:N, :] + hp_ref[...]) \
        * dis_ref[...] + b_ref[...]
    m = jnp.mean(pre, axis=0, keepdims=True)
    v = jnp.mean((pre - m) ** 2, axis=0, keepdims=True)
    act = jnp.maximum((pre - m) * lax.rsqrt(v + 1e-5) * g_ref[...]
                      + be_ref[...], 0.0)
    hm = jnp.mean(act, axis=0, keepdims=True)
    o_ref[...] = jnp.dot(hm, wfc_ref[...],
                         preferred_element_type=jnp.float32) + bfc_ref[...]


def _t3(sp, hp, dis, b, g, be, wfc, bfc):
    cdim = wfc.shape[1]
    return pl.pallas_call(
        _t3_body,
        out_shape=jax.ShapeDtypeStruct((1, cdim), jnp.float32),
    )(sp, hp, dis, b.reshape(1, H), g.reshape(1, H), be.reshape(1, H),
      wfc, bfc.reshape(1, cdim))


def kernel(x, edge_index, W1, b1, g1, be1, W2, b2, g2, be2,
           W3, b3, g3, be3, Wfc, bfc):
    srcw = edge_index[0].reshape(NW, EPW)
    dstw = edge_index[1].reshape(NW, EPW)
    srcw, dstw = jax.lax.sort_key_val(srcw, dstw, dimension=1)
    src2d = jnp.concatenate(
        [srcw, jnp.zeros((NW, PAD), jnp.int32)], axis=1).reshape(NW * NCH, CK)
    dst2d = jnp.concatenate(
        [dstw, jnp.full((NW, PAD), N, jnp.int32)], axis=1).reshape(NW * NCH, CK)

    segsum_kernel = _sc_kernels()
    ones_n = jnp.ones((N, H), jnp.float32)
    lin = (jnp.arange(NW * NCH, dtype=jnp.int32)[:, None] % 78) * CK \
        + jnp.arange(CK, dtype=jnp.int32)[None, :]
    deg_parts = segsum_kernel(ones_n, lin, dst2d)
    h1 = _matmul(x, W1, 1000)
    dis, h1p = _t1(deg_parts, h1)

    s1 = segsum_kernel(h1p, src2d, dst2d)
    h2p = _t2(s1, h1p, dis, b1, g1, be1, W2)
    s2 = segsum_kernel(h2p, src2d, dst2d)
    h3p = _t2(s2, h2p, dis, b2, g2, be2, W3)
    s3 = segsum_kernel(h3p, src2d, dst2d)
    return _t3(s3, h3p, dis, b3, g3, be3, Wfc, bfc)


# speedup vs baseline: 1.1988x; 1.1988x over previous
import functools

import jax
import jax.numpy as jnp
from jax import lax
from jax.experimental import pallas as pl
from jax.experimental.pallas import tpu as pltpu
from jax.experimental.pallas import tpu_sc as plsc

N = 10000
E = 160000
H = 128
NC = 2
NS = 16
NW = NC * NS
EPW = E // NW
CK = 128
NCH = 40
PAD = NCH * CK - EPW
NP = 10240
RPS = NP // NS
ZR = 128


def _mesh():
    return plsc.VectorSubcoreMesh(core_axis_name="c", subcore_axis_name="s",
                                  num_cores=NC, num_subcores=NS)


def _segsum_body(hp_hbm, src_hbm, dst_hbm, out_hbm,
                 src_v, dst_v, buf0, buf1, gsem0, gsem1, acc_sh):
    c = lax.axis_index("c")
    s = lax.axis_index("s")
    w = c * NS + s

    pltpu.sync_copy(src_hbm.at[pl.ds(w * NCH, NCH)], src_v)
    pltpu.sync_copy(dst_hbm.at[pl.ds(w * NCH, NCH)], dst_v)

    def fill(r, carry):
        for t in range(H // 16):
            buf0[r, pl.ds(t * 16, 16)] = jnp.zeros((16,), jnp.float32)
        return carry
    lax.fori_loop(0, ZR, fill, 0)

    def _wait(ref, sem):
        pltpu.make_async_copy(hp_hbm.at[pl.ds(0, CK)], ref, sem).wait()

    for t in range(RPS // ZR):
        pltpu.async_copy(buf0, acc_sh.at[pl.ds(s * RPS + t * ZR, ZR)], gsem0)
    for t in range(RPS // ZR):
        _wait(buf0, gsem0)
    pltpu.async_copy(hp_hbm.at[src_v.at[0]], buf0, gsem0)
    pltpu.async_copy(hp_hbm.at[src_v.at[1]], buf1, gsem1)
    plsc.subcore_barrier()

    def pair(j2, carry):
        j = 2 * j2
        _wait(buf0, gsem0)
        pltpu.sync_copy(buf0, acc_sh.at[dst_v.at[j]], add=True)

        @pl.when(j + 2 < NCH)
        def _():
            pltpu.async_copy(hp_hbm.at[src_v.at[j + 2]], buf0, gsem0)
        _wait(buf1, gsem1)
        pltpu.sync_copy(buf1, acc_sh.at[dst_v.at[j + 1]], add=True)

        @pl.when(j + 3 < NCH)
        def _():
            pltpu.async_copy(hp_hbm.at[src_v.at[j + 3]], buf1, gsem1)
        return carry
    lax.fori_loop(0, NCH // 2, pair, 0)
    plsc.subcore_barrier()

    pltpu.sync_copy(acc_sh.at[pl.ds(s * RPS, RPS)],
                    out_hbm.at[c, pl.ds(s * RPS, RPS)])


@functools.lru_cache(maxsize=None)
def _sc_kernels():
    segsum = pl.kernel(
        _segsum_body,
        out_type=jax.ShapeDtypeStruct((NC, NP, H), jnp.float32),
        mesh=_mesh(),
        scratch_types=[
            pltpu.VMEM((NCH, CK), jnp.int32),
            pltpu.VMEM((NCH, CK), jnp.int32),
            pltpu.VMEM((CK, H), jnp.float32),
            pltpu.VMEM((CK, H), jnp.float32),
            pltpu.SemaphoreType.DMA,
            pltpu.SemaphoreType.DMA,
            pltpu.VMEM_SHARED((NP, H), jnp.float32),
        ],
    )
    return segsum


def _mm_body(x_ref, w_ref, o_ref):
    o_ref[...] = jnp.dot(x_ref[...], w_ref[...],
                         preferred_element_type=jnp.float32)


def _matmul(x, w, bm):
    m, k = x.shape
    n = w.shape[1]
    return pl.pallas_call(
        _mm_body,
        grid=(m // bm,),
        in_specs=[pl.BlockSpec((bm, k), lambda i: (i, 0)),
                  pl.BlockSpec((k, n), lambda i: (0, 0))],
        out_specs=pl.BlockSpec((bm, n), lambda i: (i, 0)),
        out_shape=jax.ShapeDtypeStruct((m, n), jnp.float32),
    )(x, w)


def _t1_body(dp_ref, h1_ref, dis_ref, h1p_ref):
    deg = 1.0 + dp_ref[0, 0:N, 0:1] + dp_ref[1, 0:N, 0:1]
    dis = lax.rsqrt(deg)
    dis_ref[...] = dis
    h1p_ref[...] = h1_ref[...] * dis


def _t1(deg_parts, h1):
    return pl.pallas_call(
        _t1_body,
        out_shape=[jax.ShapeDtypeStruct((N, 1), jnp.float32),
                   jax.ShapeDtypeStruct((N, H), jnp.float32)],
    )(deg_parts, h1)


def _t2_body(sp_ref, hp_ref, dis_ref, b_ref, g_ref, be_ref, w_ref, o_ref):
    dis = dis_ref[...]
    pre = (sp_ref[0, 0:N, :] + sp_ref[1, 0:N, :] + hp_ref[...]) * dis \
        + b_ref[...]
    m = jnp.mean(pre, axis=0, keepdims=True)
    v = jnp.mean((pre - m) ** 2, axis=0, keepdims=True)
    act = jnp.maximum((pre - m) * lax.rsqrt(v + 1e-5) * g_ref[...]
                      + be_ref[...], 0.0)
    o_ref[...] = jnp.dot(act, w_ref[...],
                         preferred_element_type=jnp.float32) * dis


def _t2(sp, hp, dis, b, g, be, w_next):
    return pl.pallas_call(
        _t2_body,
        out_shape=jax.ShapeDtypeStruct((N, H), jnp.float32),
    )(sp, hp, dis, b.reshape(1, H), g.reshape(1, H), be.reshape(1, H),
      w_next)


def _t3_body(sp_ref, hp_ref, dis_ref, b_ref, g_ref, be_ref, wfc_ref,
             bfc_ref, o_ref):
    pre = (sp_ref[0, 0:N, :] + sp_ref[1, 0:N, :] + hp_ref[...]) \
        * dis_ref[...] + b_ref[...]
    m = jnp.mean(pre, axis=0, keepdims=True)
    v = jnp.mean((pre - m) ** 2, axis=0, keepdims=True)
    act = jnp.maximum((pre - m) * lax.rsqrt(v + 1e-5) * g_ref[...]
                      + be_ref[...], 0.0)
    hm = jnp.mean(act, axis=0, keepdims=True)
    o_ref[...] = jnp.dot(hm, wfc_ref[...],
                         preferred_element_type=jnp.float32) + bfc_ref[...]


def _t3(sp, hp, dis, b, g, be, wfc, bfc):
    cdim = wfc.shape[1]
    return pl.pallas_call(
        _t3_body,
        out_shape=jax.ShapeDtypeStruct((1, cdim), jnp.float32),
    )(sp, hp, dis, b.reshape(1, H), g.reshape(1, H), be.reshape(1, H),
      wfc, bfc.reshape(1, cdim))


def kernel(x, edge_index, W1, b1, g1, be1, W2, b2, g2, be2,
           W3, b3, g3, be3, Wfc, bfc):
    srcw = edge_index[0].reshape(NW, EPW)
    dstw = edge_index[1].reshape(NW, EPW)
    src2d = jnp.concatenate(
        [srcw, jnp.zeros((NW, PAD), jnp.int32)], axis=1).reshape(NW * NCH, CK)
    dst2d = jnp.concatenate(
        [dstw, jnp.full((NW, PAD), N, jnp.int32)], axis=1).reshape(NW * NCH, CK)

    segsum_kernel = _sc_kernels()
    ones_n = jnp.ones((N, H), jnp.float32)
    lin = (jnp.arange(NW * NCH, dtype=jnp.int32)[:, None] % 78) * CK \
        + jnp.arange(CK, dtype=jnp.int32)[None, :]
    deg_parts = segsum_kernel(ones_n, lin, dst2d)
    h1 = _matmul(x, W1, 1000)
    dis, h1p = _t1(deg_parts, h1)

    s1 = segsum_kernel(h1p, src2d, dst2d)
    h2p = _t2(s1, h1p, dis, b1, g1, be1, W2)
    s2 = segsum_kernel(h2p, src2d, dst2d)
    h3p = _t2(s2, h2p, dis, b2, g2, be2, W3)
    s3 = segsum_kernel(h3p, src2d, dst2d)
    return _t3(s3, h3p, dis, b3, g3, be3, Wfc, bfc)
